# baseline (speedup 1.0000x reference)
import functools

import jax
import jax.numpy as jnp
from jax.experimental import pallas as pl
from jax.experimental.pallas import tpu as pltpu

_BM = 200


def _body(x_ref, l_ref, u_ref, wi_ref, ws_ref, out_ref, xb_ref):
    m = pl.program_id(0)

    @pl.when(m == 0)
    def _cast_x_once():
        xb_ref[...] = x_ref[...].astype(jnp.bfloat16)

    xb = xb_ref[...]
    lb = l_ref[...].astype(jnp.bfloat16)
    ub = u_ref[...].astype(jnp.bfloat16)
    t_l = jnp.dot(lb, xb, preferred_element_type=jnp.float32)
    t_u = jnp.dot(ub, xb, preferred_element_type=jnp.float32)
    t = (jnp.dot(t_l, wi_ref[...], preferred_element_type=jnp.float32)
         + jnp.dot(t_u, ws_ref[...], preferred_element_type=jnp.float32))
    out_ref[pl.ds(m * l_ref.shape[0], l_ref.shape[0]), :] = jnp.maximum(t, 0.0)


def _run(x, lower, upper, w_irr, w_sol, bm):
    n, d = x.shape
    d_out = w_irr.shape[1]
    return pl.pallas_call(
        _body,
        grid=(n // bm,),
        in_specs=[
            pl.BlockSpec((n, d), lambda m: (0, 0)),
            pl.BlockSpec((bm, n), lambda m: (m, 0)),
            pl.BlockSpec((bm, n), lambda m: (m, 0)),
            pl.BlockSpec((d, d_out), lambda m: (0, 0)),
            pl.BlockSpec((d, d_out), lambda m: (0, 0)),
        ],
        out_specs=pl.BlockSpec((n, d_out), lambda m: (0, 0)),
        out_shape=jax.ShapeDtypeStruct((n, d_out), jnp.float32),
        scratch_shapes=[pltpu.VMEM((n, d), jnp.bfloat16)],
        compiler_params=pltpu.CompilerParams(
            dimension_semantics=("parallel",),
        ),
    )(x, lower, upper, w_irr, w_sol)


def kernel(x, lower_neighborhood, upper_neighborhood, W_irr, W_sol):
    return _run(x, lower_neighborhood, upper_neighborhood, W_irr, W_sol, _BM)


# speedup vs baseline: 1.0010x; 1.0010x over previous
import functools

import jax
import jax.numpy as jnp
from jax.experimental import pallas as pl
from jax.experimental.pallas import tpu as pltpu

_BM = 200
_NBUF = 3


def _body(x_ref, l_hbm, u_hbm, wi_ref, ws_ref, out_hbm,
          xb_ref, lbuf, ubuf, obuf, lsem, usem, osem, *, bm):
    n, d = x_ref.shape
    nsteps = n // bm
    xb_ref[...] = x_ref[...].astype(jnp.bfloat16)

    def in_copies(i):
        s = i % _NBUF
        return (
            pltpu.make_async_copy(
                l_hbm.at[pl.ds(i * bm, bm), :], lbuf.at[s], lsem.at[s]),
            pltpu.make_async_copy(
                u_hbm.at[pl.ds(i * bm, bm), :], ubuf.at[s], usem.at[s]),
        )

    def out_copy(i):
        s = i % 2
        return pltpu.make_async_copy(
            obuf.at[s], out_hbm.at[pl.ds(i * bm, bm), :], osem.at[s])

    for i in range(_NBUF):
        for c in in_copies(i):
            c.start()

    for i in range(nsteps):
        s = i % _NBUF
        for c in in_copies(i):
            c.wait()
        lb = lbuf[s].astype(jnp.bfloat16)
        ub = ubuf[s].astype(jnp.bfloat16)
        t_l = jnp.dot(lb, xb_ref[...], preferred_element_type=jnp.float32)
        t_u = jnp.dot(ub, xb_ref[...], preferred_element_type=jnp.float32)
        t = (jnp.dot(t_l, wi_ref[...], preferred_element_type=jnp.float32)
             + jnp.dot(t_u, ws_ref[...], preferred_element_type=jnp.float32))
        if i >= 2:
            out_copy(i - 2).wait()
        obuf[i % 2] = jnp.maximum(t, 0.0)
        out_copy(i).start()
        if i + _NBUF < nsteps:
            for c in in_copies(i + _NBUF):
                c.start()

    out_copy(nsteps - 2).wait()
    out_copy(nsteps - 1).wait()


def _run(x, lower, upper, w_irr, w_sol, bm):
    n, d = x.shape
    d_out = w_irr.shape[1]
    return pl.pallas_call(
        functools.partial(_body, bm=bm),
        in_specs=[
            pl.BlockSpec(memory_space=pltpu.MemorySpace.VMEM),
            pl.BlockSpec(memory_space=pltpu.MemorySpace.HBM),
            pl.BlockSpec(memory_space=pltpu.MemorySpace.HBM),
            pl.BlockSpec(memory_space=pltpu.MemorySpace.VMEM),
            pl.BlockSpec(memory_space=pltpu.MemorySpace.VMEM),
        ],
        out_specs=pl.BlockSpec(memory_space=pltpu.MemorySpace.HBM),
        out_shape=jax.ShapeDtypeStruct((n, d_out), jnp.float32),
        scratch_shapes=[
            pltpu.VMEM((n, d), jnp.bfloat16),
            pltpu.VMEM((_NBUF, bm, n), jnp.float32),
            pltpu.VMEM((_NBUF, bm, n), jnp.float32),
            pltpu.VMEM((2, bm, d_out), jnp.float32),
            pltpu.SemaphoreType.DMA((_NBUF,)),
            pltpu.SemaphoreType.DMA((_NBUF,)),
            pltpu.SemaphoreType.DMA((2,)),
        ],
    )(x, lower, upper, w_irr, w_sol)


def kernel(x, lower_neighborhood, upper_neighborhood, W_irr, W_sol):
    return _run(x, lower_neighborhood, upper_neighborhood, W_irr, W_sol, _BM)


# speedup vs baseline: 1.0040x; 1.0030x over previous
import jax
import jax.numpy as jnp
from jax.experimental import pallas as pl
from jax.experimental.pallas import tpu as pltpu

_BM = 200


def _body(x_ref, l_ref, u_ref, wi_ref, ws_ref, out_ref):
    xb = x_ref[...].astype(jnp.bfloat16)
    lb = l_ref[...].astype(jnp.bfloat16)
    ub = u_ref[...].astype(jnp.bfloat16)
    t_l = jnp.dot(lb, xb, preferred_element_type=jnp.float32)
    t_u = jnp.dot(ub, xb, preferred_element_type=jnp.float32)
    t = (jnp.dot(t_l, wi_ref[...], preferred_element_type=jnp.float32)
         + jnp.dot(t_u, ws_ref[...], preferred_element_type=jnp.float32))
    out_ref[...] = jnp.maximum(t, 0.0)


def _run(x, lower, upper, w_irr, w_sol, bm):
    n, d = x.shape
    d_out = w_irr.shape[1]
    return pl.pallas_call(
        _body,
        grid=(n // bm,),
        in_specs=[
            pl.BlockSpec((n, d), lambda m: (0, 0)),
            pl.BlockSpec((bm, n), lambda m: (m, 0)),
            pl.BlockSpec((bm, n), lambda m: (m, 0)),
            pl.BlockSpec((d, d_out), lambda m: (0, 0)),
            pl.BlockSpec((d, d_out), lambda m: (0, 0)),
        ],
        out_specs=pl.BlockSpec((bm, d_out), lambda m: (m, 0)),
        out_shape=jax.ShapeDtypeStruct((n, d_out), jnp.float32),
        compiler_params=pltpu.CompilerParams(
            dimension_semantics=("parallel",),
        ),
    )(x, lower, upper, w_irr, w_sol)


def kernel(x, lower_neighborhood, upper_neighborhood, W_irr, W_sol):
    return _run(x, lower_neighborhood, upper_neighborhood, W_irr, W_sol, _BM)
